# baseline (speedup 1.0000x reference)
import jax
import jax.numpy as jnp
from jax.experimental import pallas as pl
from jax.experimental.pallas import tpu as pltpu


def _gcn_block_kernel(seq_ref, adj_ref, w_ref, b_ref, a_ref, out_ref, sf_ref):
    @pl.when(pl.program_id(0) == 0)
    def _():
        sf_ref[...] = jnp.dot(
            seq_ref[...], w_ref[...].T, preferred_element_type=jnp.float32
        )

    o = jnp.dot(adj_ref[...], sf_ref[...], preferred_element_type=jnp.float32)
    o = o + b_ref[...]
    a = a_ref[0, 0]
    out_ref[...] = jnp.where(o >= 0, o, a * o)


def kernel(seq, adj, W, bias, prelu_a):
    n, in_ft = seq.shape
    out_ft = W.shape[0]
    bm = 200
    grid = (pl.cdiv(n, bm),)

    return pl.pallas_call(
        _gcn_block_kernel,
        grid=grid,
        in_specs=[
            pl.BlockSpec((n, in_ft), lambda i: (0, 0)),
            pl.BlockSpec((bm, n), lambda i: (i, 0)),
            pl.BlockSpec((out_ft, in_ft), lambda i: (0, 0)),
            pl.BlockSpec((1, out_ft), lambda i: (0, 0)),
            pl.BlockSpec(memory_space=pltpu.SMEM),
        ],
        out_specs=pl.BlockSpec((bm, out_ft), lambda i: (i, 0)),
        out_shape=jax.ShapeDtypeStruct((n, out_ft), jnp.float32),
        scratch_shapes=[pltpu.VMEM((n, out_ft), jnp.float32)],
    )(seq, adj, W, bias.reshape(1, out_ft), prelu_a.reshape(1, 1))


# speedup vs baseline: 1.0074x; 1.0074x over previous
import jax
import jax.numpy as jnp
from jax.experimental import pallas as pl
from jax.experimental.pallas import tpu as pltpu


def _gcn_block_kernel(seq_ref, adj_ref, w_ref, b_ref, a_ref, out_ref, sf_ref):
    @pl.when(pl.program_id(0) == 0)
    def _():
        sf_ref[...] = jnp.dot(
            seq_ref[...], w_ref[...].T, preferred_element_type=jnp.float32
        )

    o = jnp.dot(adj_ref[...], sf_ref[...], preferred_element_type=jnp.float32)
    o = o + b_ref[...]
    a = a_ref[0, 0]
    out_ref[...] = jnp.where(o >= 0, o, a * o)


def kernel(seq, adj, W, bias, prelu_a):
    n, in_ft = seq.shape
    out_ft = W.shape[0]
    bm = 256
    grid = (pl.cdiv(n, bm),)

    return pl.pallas_call(
        _gcn_block_kernel,
        grid=grid,
        in_specs=[
            pl.BlockSpec((n, in_ft), lambda i: (0, 0)),
            pl.BlockSpec((bm, n), lambda i: (i, 0)),
            pl.BlockSpec((out_ft, in_ft), lambda i: (0, 0)),
            pl.BlockSpec((1, out_ft), lambda i: (0, 0)),
            pl.BlockSpec(memory_space=pltpu.SMEM),
        ],
        out_specs=pl.BlockSpec((bm, out_ft), lambda i: (i, 0)),
        out_shape=jax.ShapeDtypeStruct((n, out_ft), jnp.float32),
        scratch_shapes=[pltpu.VMEM((n, out_ft), jnp.float32)],
    )(seq, adj, W, bias.reshape(1, out_ft), prelu_a.reshape(1, 1))


# speedup vs baseline: 1.0171x; 1.0097x over previous
import jax
import jax.numpy as jnp
from jax.experimental import pallas as pl
from jax.experimental.pallas import tpu as pltpu


def _gcn_block_kernel(seq_ref, adj_ref, w_ref, b_ref, a_ref, out_ref, sf_ref):
    @pl.when(pl.program_id(0) == 0)
    def _():
        sf_ref[...] = jnp.dot(
            seq_ref[...], w_ref[...].T, preferred_element_type=jnp.float32
        )

    o = jnp.dot(adj_ref[...], sf_ref[...], preferred_element_type=jnp.float32)
    o = o + b_ref[...]
    a = a_ref[0, 0]
    out_ref[...] = jnp.where(o >= 0, o, a * o)


def kernel(seq, adj, W, bias, prelu_a):
    n, in_ft = seq.shape
    out_ft = W.shape[0]
    bm = 240
    grid = (pl.cdiv(n, bm),)

    return pl.pallas_call(
        _gcn_block_kernel,
        grid=grid,
        in_specs=[
            pl.BlockSpec((n, in_ft), lambda i: (0, 0)),
            pl.BlockSpec((bm, n), lambda i: (i, 0)),
            pl.BlockSpec((out_ft, in_ft), lambda i: (0, 0)),
            pl.BlockSpec((1, out_ft), lambda i: (0, 0)),
            pl.BlockSpec(memory_space=pltpu.SMEM),
        ],
        out_specs=pl.BlockSpec((bm, out_ft), lambda i: (i, 0)),
        out_shape=jax.ShapeDtypeStruct((n, out_ft), jnp.float32),
        scratch_shapes=[pltpu.VMEM((n, out_ft), jnp.float32)],
    )(seq, adj, W, bias.reshape(1, out_ft), prelu_a.reshape(1, 1))
